# Initial kernel scaffold; baseline (speedup 1.0000x reference)
import jax
import jax.numpy as jnp
from jax.experimental import pallas as pl


def kernel(users_feature, exercises_feature, knowledge_feature, W_mlp, b_mlp, ue_src, ue_dst, ek_src, ek_dst):
    raise NotImplementedError("write your pallas kernel here")


# speedup vs baseline: 8.3567x; 8.3567x over previous
import functools

import jax
import jax.numpy as jnp
from jax import lax
from jax.experimental import pallas as pl
from jax.experimental.pallas import tpu as pltpu
from jax.experimental.pallas import tpu_sc as plsc

NU = 50000
NE = 50000
NK = 2048
D = 64
NUM_LAYERS = 2

NC = 2
NS = 16
LANE = 16

NA_PAD = 50176
N1_PAD = 100352
N2_PAD = 53248
E1_PAD = 802816
E2_PAD = 212992

_EPS = 1e-8


def _make_deg_kernel(n_pad, e_pad, off_b):
  ept = e_pad // (NC * NS)
  nblk = ept // 512
  mesh = plsc.VectorSubcoreMesh(core_axis_name="c", subcore_axis_name="s")

  @functools.partial(
      pl.kernel,
      out_type=jax.ShapeDtypeStruct((NC * NS, n_pad), jnp.float32),
      mesh=mesh,
      compiler_params=pltpu.CompilerParams(needs_layout_passes=False, use_tc_tiling_on_sc=False),
      scratch_types=[
          pltpu.VMEM((n_pad,), jnp.float32),
          pltpu.VMEM((512,), jnp.int32),
          pltpu.VMEM((512,), jnp.int32),
      ],
  )
  def deg_kernel(src_hbm, dst_hbm, zdeg_hbm, out_hbm, hist, sbuf, dbuf):
    c = lax.axis_index("c")
    s = lax.axis_index("s")
    wid = s * NC + c
    ones = jnp.ones((LANE,), jnp.float32)
    offv = jnp.full((LANE,), off_b, jnp.int32)

    pltpu.sync_copy(zdeg_hbm.at[pl.ds(0, n_pad)], hist)

    base = wid * ept

    def blk_body(blk, carry):
      off = base + blk * 512
      pltpu.sync_copy(src_hbm.at[pl.ds(off, 512)], sbuf)
      pltpu.sync_copy(dst_hbm.at[pl.ds(off, 512)], dbuf)
      for k in range(32):
        v = sbuf[pl.ds(k * LANE, LANE)]
        plsc.addupdate_scatter(hist, [v], ones)
        w = dbuf[pl.ds(k * LANE, LANE)] + offv
        plsc.addupdate_scatter(hist, [w], ones)
      return carry

    lax.fori_loop(0, nblk, blk_body, 0)
    pltpu.sync_copy(hist, out_hbm.at[wid])

  return deg_kernel


def _make_spmm_kernel(n_gather, n_out, e_pad):
  half = n_out // 2
  rows_pt = half // NS
  acc_rows = half + 128
  zrows = acc_rows // NS
  ept = e_pad // NS
  nblk = ept // 1024
  mesh = plsc.VectorSubcoreMesh(core_axis_name="c", subcore_axis_name="s")

  @functools.partial(
      pl.kernel,
      out_type=jax.ShapeDtypeStruct((n_out, D), jnp.float32),
      mesh=mesh,
      compiler_params=pltpu.CompilerParams(needs_layout_passes=False, use_tc_tiling_on_sc=False),
      scratch_types=[
          pltpu.VMEM((1024,), jnp.int32),
          pltpu.VMEM((1024,), jnp.int32),
          pltpu.VMEM((8, 128), jnp.int32),
          pltpu.VMEM((2, 128, D), jnp.float32),
          pltpu.VMEM_SHARED((acc_rows, D), jnp.float32),
          pltpu.SemaphoreType.DMA,
          pltpu.SemaphoreType.DMA,
      ],
  )
  def spmm_kernel(g_hbm, gidx_hbm, sidx_hbm, z2d_hbm, out_hbm,
                  gidx_v, sidx_v, sloc_v, rows_v, acc, sem0, sem1):
    c = lax.axis_index("c")
    s = lax.axis_index("s")
    lo = c * half
    hi = lo + half
    trash = half + s

    pltpu.sync_copy(z2d_hbm.at[pl.ds(0, zrows)],
                    acc.at[pl.ds(s * zrows, zrows)])
    plsc.subcore_barrier()

    base = s * ept
    sems = [sem0, sem1]

    def blk_body(blk, carry):
      off = base + blk * 1024
      pltpu.sync_copy(gidx_hbm.at[pl.ds(off, 1024)], gidx_v)
      pltpu.sync_copy(sidx_hbm.at[pl.ds(off, 1024)], sidx_v)
      for j in range(8):
        for k in range(8):
          v = sidx_v[pl.ds(j * 128 + k * LANE, LANE)]
          inr = (v >= lo) & (v < hi)
          sloc_v[j, pl.ds(k * LANE, LANE)] = jnp.where(inr, v - lo, trash)
      descs = [None, None]
      descs[0] = pltpu.async_copy(
          g_hbm.at[gidx_v.at[pl.ds(0, 128)]], rows_v.at[0], sems[0])
      for j in range(8):
        if j < 7:
          b = (j + 1) % 2
          descs[b] = pltpu.async_copy(
              g_hbm.at[gidx_v.at[pl.ds((j + 1) * 128, 128)]],
              rows_v.at[b], sems[b])
        descs[j % 2].wait()
        pltpu.sync_copy(rows_v.at[j % 2], acc.at[sloc_v.at[j]], add=True)
      return carry

    lax.fori_loop(0, nblk, blk_body, 0)
    plsc.subcore_barrier()
    pltpu.sync_copy(acc.at[pl.ds(s * rows_pt, rows_pt)],
                    out_hbm.at[pl.ds(lo + s * rows_pt, rows_pt)])

  return spmm_kernel


def _s_invd_tc(degp):
  g_rows = degp.shape[1]
  blk = 16

  def body(degp_ref, s_ref, invd_ref):
    d = jnp.sum(degp_ref[...], axis=0)
    s_ref[...] = 1.0 / (jnp.sqrt(d) + _EPS)
    invd_ref[...] = 1.0 / (d + _EPS)

  return pl.pallas_call(
      body,
      grid=(g_rows // blk,),
      in_specs=[pl.BlockSpec((NC * NS, blk, 128), lambda i: (0, i, 0))],
      out_specs=[pl.BlockSpec((blk, 128), lambda i: (i, 0)),
                 pl.BlockSpec((blk, 128), lambda i: (i, 0))],
      out_shape=[jax.ShapeDtypeStruct((g_rows, 128), jnp.float32),
                 jax.ShapeDtypeStruct((g_rows, 128), jnp.float32)],
  )(degp)


def _scale_tc(feats, s_col):
  n = feats.shape[0]
  blk = 512

  def body(f_ref, s_ref, o_ref):
    o_ref[...] = f_ref[...] * s_ref[...]

  return pl.pallas_call(
      body,
      grid=(n // blk,),
      in_specs=[pl.BlockSpec((blk, D), lambda i: (i, 0)),
                pl.BlockSpec((blk, 1), lambda i: (i, 0))],
      out_specs=pl.BlockSpec((blk, D), lambda i: (i, 0)),
      out_shape=jax.ShapeDtypeStruct((n, D), jnp.float32),
  )(feats, s_col)


def _post_tc(raw, s_col, acc, layer, need_g):
  n = raw.shape[0]
  blk = 512
  damp = 1.0 / (layer + 2)

  def body(raw_ref, s_ref, acc_ref, accout_ref, *maybe_g):
    f = raw_ref[...] * s_ref[...] * damp
    l2 = jnp.sqrt(jnp.sum(f * f, axis=1, keepdims=True))
    accout_ref[...] = acc_ref[...] + f / jnp.maximum(l2, 1e-12)
    if need_g:
      maybe_g[0][...] = f * s_ref[...]

  out_shape = [jax.ShapeDtypeStruct((n, D), jnp.float32)]
  out_specs = [pl.BlockSpec((blk, D), lambda i: (i, 0))]
  if need_g:
    out_shape.append(jax.ShapeDtypeStruct((n, D), jnp.float32))
    out_specs.append(pl.BlockSpec((blk, D), lambda i: (i, 0)))

  res = pl.pallas_call(
      body,
      grid=(n // blk,),
      in_specs=[pl.BlockSpec((blk, D), lambda i: (i, 0)),
                pl.BlockSpec((blk, 1), lambda i: (i, 0)),
                pl.BlockSpec((blk, D), lambda i: (i, 0))],
      out_specs=out_specs,
      out_shape=out_shape,
  )(raw, s_col, acc)
  if need_g:
    return res[0], res[1]
  return res[0], None


def _mlp_tc(eu, ek, eagg, invd_col, w1, w2, bias):
  n = eu.shape[0]
  blk = 512

  def body(eu_ref, ek_ref, ea_ref, iv_ref, w1_ref, w2_ref, b_ref, o_ref):
    h2 = ek_ref[...] + ea_ref[...] * iv_ref[...]
    o = (jnp.dot(eu_ref[...], w1_ref[...],
                 preferred_element_type=jnp.float32)
         + jnp.dot(h2, w2_ref[...], preferred_element_type=jnp.float32)
         + b_ref[...])
    o_ref[...] = jnp.maximum(o, 0.0)

  return pl.pallas_call(
      body,
      grid=(n // blk,),
      in_specs=[pl.BlockSpec((blk, D), lambda i: (i, 0)),
                pl.BlockSpec((blk, D), lambda i: (i, 0)),
                pl.BlockSpec((blk, D), lambda i: (i, 0)),
                pl.BlockSpec((blk, 1), lambda i: (i, 0)),
                pl.BlockSpec((D, D), lambda i: (0, 0)),
                pl.BlockSpec((D, D), lambda i: (0, 0)),
                pl.BlockSpec((1, D), lambda i: (0, 0))],
      out_specs=pl.BlockSpec((blk, D), lambda i: (i, 0)),
      out_shape=jax.ShapeDtypeStruct((n, D), jnp.float32),
  )(eu, ek, eagg, invd_col, w1, w2, bias)


_BIG = 0x0FFFFFFF


def _padi(x, n, val):
  return jnp.pad(x, (0, n - x.shape[0]), constant_values=val)


def _padf(x, n):
  return jnp.pad(x, ((0, n - x.shape[0]), (0, 0)))


def _propagate(feat_a, feat_b, src, dst, deg_kernel, spmm_a, spmm_b,
               n_a, e_pad, n_pad, zdeg, z2d):
  na_pad = feat_a.shape[0]
  nb_pad = feat_b.shape[0]
  src_deg = _padi(src, e_pad, n_pad - 1)
  dst_deg = _padi(dst, e_pad, n_pad - 1 - n_a)
  g_a = _padi(src, e_pad, 0)
  s_a = _padi(src, e_pad, _BIG)
  g_b = _padi(dst, e_pad, 0)
  s_b = _padi(dst, e_pad, _BIG)

  degp = deg_kernel(src_deg, dst_deg, zdeg)
  s2d, invd2d = _s_invd_tc(degp.reshape(NC * NS, n_pad // 128, 128))
  s_flat = s2d.reshape(-1)
  sa_col = s_flat[0:na_pad][:, None]
  sb_col = s_flat[n_a:n_a + nb_pad][:, None]

  acc_a, acc_b = feat_a, feat_b
  ga = _scale_tc(feat_a, sa_col)
  gb = _scale_tc(feat_b, sb_col)
  for i in range(NUM_LAYERS):
    raw_a = spmm_a(gb, g_b, s_a, z2d)
    raw_b = spmm_b(ga, g_a, s_b, z2d)
    need_g = i < NUM_LAYERS - 1
    acc_a, ga = _post_tc(raw_a, sa_col, acc_a, i, need_g)
    acc_b, gb = _post_tc(raw_b, sb_col, acc_b, i, need_g)
  return acc_a, acc_b, invd2d


def kernel(users_feature, exercises_feature, knowledge_feature, W_mlp, b_mlp,
           ue_src, ue_dst, ek_src, ek_dst):
  z2d = jnp.zeros((1600, D), jnp.float32)
  zdeg = jnp.zeros((N1_PAD,), jnp.float32)

  ua = _padf(users_feature, NA_PAD)
  ea = _padf(exercises_feature, NA_PAD)
  ka = knowledge_feature

  deg1 = _make_deg_kernel(N1_PAD, E1_PAD, NU)
  deg2 = _make_deg_kernel(N2_PAD, E2_PAD, NE)
  spmm_1a = _make_spmm_kernel(NA_PAD, NA_PAD, E1_PAD)
  spmm_2a = _make_spmm_kernel(NK, NA_PAD, E2_PAD)
  spmm_2b = _make_spmm_kernel(NA_PAD, NK, E2_PAD)

  u_acc, eu_acc, _ = _propagate(
      ua, ea, ue_src, ue_dst, deg1, spmm_1a, spmm_1a,
      NU, E1_PAD, N1_PAD, zdeg, z2d)

  ek_acc, k_acc, invd2 = _propagate(
      ea, ka, ek_src, ek_dst, deg2, spmm_2a, spmm_2b,
      NE, E2_PAD, N2_PAD, zdeg[:N2_PAD], z2d)

  eagg_raw = spmm_2a(k_acc, _padi(ek_dst, E2_PAD, 0),
                     _padi(ek_src, E2_PAD, _BIG), z2d)
  invd_e = invd2.reshape(-1)[0:NA_PAD][:, None]

  e_rep = _mlp_tc(eu_acc, ek_acc, eagg_raw, invd_e,
                  W_mlp[:D], W_mlp[D:], b_mlp[None, :])
  return jnp.concatenate([u_acc[:NU], e_rep[:NE]], axis=0)
